# baseline (speedup 1.0000x reference)
import functools

import jax
import jax.numpy as jnp
from jax import lax
from jax.experimental import pallas as pl
from jax.experimental.pallas import tpu as pltpu
from jax.experimental.pallas import tpu_sc as plsc

_VOCAB = 1000
_BATCH = 1024
_SEQ = 50
_D = _VOCAB
_N = _BATCH * _SEQ
_NSC = _N // 2
_NW = 32
_ROWS_PER_W = _NSC // _NW
_K = 25
_NCHUNK = _ROWS_PER_W // _K

_VP = 1024
_BLK = 512
_TCBLK0 = _NSC // _BLK
_NTCBLK = (_N - _NSC) // _BLK


def _make_sc_gather():
    mesh = plsc.VectorSubcoreMesh(core_axis_name="c", subcore_axis_name="s")

    @functools.partial(
        pl.kernel,
        mesh=mesh,
        compiler_params=pltpu.CompilerParams(use_tc_tiling_on_sc=False),
        out_type=jax.ShapeDtypeStruct((_N, _D), jnp.float32),
        scratch_types=[
            pltpu.VMEM((_NCHUNK, _K), jnp.int32),
            pltpu.VMEM((_K, _D), jnp.float32),
            pltpu.VMEM((_K, _D), jnp.float32),
            pltpu.VMEM_SHARED((_VOCAB, _D), jnp.float32),
            pltpu.SemaphoreType.DMA,
            pltpu.SemaphoreType.DMA,
            pltpu.SemaphoreType.DMA,
            pltpu.SemaphoreType.DMA,
        ],
    )
    def body(table_hbm, idx_hbm, out_hbm, idx_v, rows0, rows1, tab_sp,
             g0, g1, s0, s1):
        sid = lax.axis_index("s")
        wid = sid * 2 + lax.axis_index("c")
        base = wid * _ROWS_PER_W
        pltpu.sync_copy(idx_hbm.at[wid], idx_v)

        pltpu.sync_copy(table_hbm.at[pl.ds(sid * 62, 62)],
                        tab_sp.at[pl.ds(sid * 62, 62)])

        @pl.when(sid == 0)
        def _():
            pltpu.sync_copy(table_hbm.at[pl.ds(992, 8)],
                            tab_sp.at[pl.ds(992, 8)])

        plsc.subcore_barrier()

        rows = (rows0, rows1)
        gsem = (g0, g1)
        ssem = (s0, s1)

        def gather(g, b):
            return pltpu.make_async_copy(
                tab_sp.at[idx_v.at[g]], rows[b], gsem[b])

        def store(g, b):
            return pltpu.make_async_copy(
                rows[b], out_hbm.at[pl.ds(base + g * _K, _K)], ssem[b])

        gather(0, 0).start()
        gather(0, 0).wait()
        gather(1, 1).start()
        store(0, 0).start()

        def half_step(g, b):
            gather(g, b).wait()
            store(g - 1, 1 - b).wait()
            gather(g + 1, 1 - b).start()
            store(g, b).start()

        def pair(j, carry):
            i = 2 * j + 1
            half_step(i, 1)
            half_step(i + 1, 0)
            return carry

        lax.fori_loop(0, (_NCHUNK - 2) // 2, pair, 0)

        g = _NCHUNK - 1
        gather(g, 1).wait()
        store(g - 1, 0).wait()
        store(g, 1).start()
        store(g, 1).wait()

    return body


_sc_gather = _make_sc_gather()


def _tc_body(idx_ref, table_ref, alias_ref, out_ref):
    del alias_ref
    idx = idx_ref[...]
    cols = jax.lax.broadcasted_iota(jnp.int32, (_BLK, _VP), 1)
    onehot = jnp.where(idx == cols, 1.0, 0.0).astype(jnp.bfloat16)
    out_ref[...] = jax.lax.dot_general(
        onehot, table_ref[...],
        (((1,), (0,)), ((), ())),
        preferred_element_type=jnp.float32)


def _tc_fill(idx2d, table_bf16, out_sc):
    return pl.pallas_call(
        _tc_body,
        grid=(_NTCBLK,),
        in_specs=[
            pl.BlockSpec((_BLK, 1), lambda i: (_TCBLK0 + i, 0)),
            pl.BlockSpec((_VP, _D), lambda i: (0, 0)),
            pl.BlockSpec(memory_space=pl.ANY),
        ],
        out_specs=pl.BlockSpec((_BLK, _D), lambda i: (_TCBLK0 + i, 0)),
        out_shape=jax.ShapeDtypeStruct((_N, _D), jnp.float32),
        input_output_aliases={2: 0},
    )(idx2d, table_bf16, out_sc)


def kernel(inputs, table):
    idx_flat = inputs.reshape(_N).astype(jnp.int32)
    idx_sc = idx_flat[:_NSC].reshape(_NW, _NCHUNK, _K)
    out_sc = _sc_gather(table, idx_sc)

    idx2d = idx_flat.reshape(_N, 1)
    table_bf16 = jnp.pad(table.astype(jnp.bfloat16),
                         ((0, _VP - _VOCAB), (0, 0)))
    out = _tc_fill(idx2d, table_bf16, out_sc)
    return (out.reshape(_BATCH, _SEQ, _VOCAB), None)
